# baseline (speedup 1.0000x reference)
import functools

import jax
import jax.numpy as jnp
from jax import lax
from jax.experimental import pallas as pl
from jax.experimental.pallas import tpu as pltpu
from jax.experimental.pallas import tpu_sc as plsc

BATCH = 16384
SEQ = 201
DIM = 64
NC, NS = 2, 16
NW = NC * NS
BBLK = 2
BPW = BATCH // NW
CHUNKS_PER_W = BPW // BBLK
NB = 2
OUTER = CHUNKS_PER_W // NB


def _make_gather():
    mesh = plsc.VectorSubcoreMesh(core_axis_name="c", subcore_axis_name="s")

    @functools.partial(
        pl.kernel,
        mesh=mesh,
        out_type=jax.ShapeDtypeStruct((BATCH, SEQ, DIM), jnp.float32),
        scratch_types=[
            [pltpu.VMEM((BBLK, SEQ), jnp.int32) for _ in range(NB)],
            [pltpu.VMEM((BBLK, SEQ, DIM), jnp.float32) for _ in range(NB)],
            [pltpu.SemaphoreType.DMA for _ in range(NB)],
            [pltpu.SemaphoreType.DMA for _ in range(NB)],
        ],
        compiler_params=pltpu.CompilerParams(use_tc_tiling_on_sc=False),
    )
    def gather_kernel(w_hbm, table_hbm, out_hbm, idx_v, rows_v, gsem, ssem):
        wid = lax.axis_index("s") * NC + lax.axis_index("c")
        bbase = wid * BPW

        def fire_gathers(b, c):
            b0 = bbase + c * BBLK
            pltpu.sync_copy(w_hbm.at[pl.ds(b0, BBLK)], idx_v[b])
            for i in range(BBLK):
                pltpu.async_copy(
                    table_hbm.at[idx_v[b].at[i]], rows_v[b].at[i], gsem[b]
                )

        def wait_gathers(b):
            for i in range(BBLK):
                pltpu.make_async_copy(
                    table_hbm.at[idx_v[b].at[i]], rows_v[b].at[i], gsem[b]
                ).wait()

        def fire_store(b, c):
            pltpu.async_copy(
                rows_v[b], out_hbm.at[pl.ds(bbase + c * BBLK, BBLK)], ssem[b]
            )

        def wait_store(b, c):
            pltpu.make_async_copy(
                rows_v[b], out_hbm.at[pl.ds(bbase + c * BBLK, BBLK)], ssem[b]
            ).wait()

        for b in range(NB):
            fire_gathers(b, b)

        def body(g, carry):
            c0 = g * NB
            for b in range(NB):
                wait_gathers(b)
                fire_store(b, c0 + b)

            @pl.when(g < OUTER - 1)
            def _prefetch():
                for b in range(NB):
                    wait_store(b, c0 + b)
                    fire_gathers(b, c0 + NB + b)

            return carry

        lax.fori_loop(0, OUTER, body, 0)
        for b in range(NB):
            wait_store(b, (OUTER - 1) * NB + b)

    return gather_kernel


_gather = _make_gather()


@jax.jit
def kernel(w, table):
    return _gather(w.astype(jnp.int32), table)


# speedup vs baseline: 1.0059x; 1.0059x over previous
import functools

import jax
import jax.numpy as jnp
from jax import lax
from jax.experimental import pallas as pl
from jax.experimental.pallas import tpu as pltpu
from jax.experimental.pallas import tpu_sc as plsc

BATCH = 16384
SEQ = 201
DIM = 64
NC, NS = 2, 16
NW = NC * NS
BBLK = 2
BPW = BATCH // NW
CHUNKS_PER_W = BPW // BBLK
NB = 4
OUTER = CHUNKS_PER_W // NB


def _make_gather():
    mesh = plsc.VectorSubcoreMesh(core_axis_name="c", subcore_axis_name="s")

    @functools.partial(
        pl.kernel,
        mesh=mesh,
        out_type=jax.ShapeDtypeStruct((BATCH, SEQ, DIM), jnp.float32),
        scratch_types=[
            [pltpu.VMEM((BBLK, SEQ), jnp.int32) for _ in range(NB)],
            [pltpu.VMEM((BBLK, SEQ, DIM), jnp.float32) for _ in range(NB)],
            [pltpu.SemaphoreType.DMA for _ in range(NB)],
            [pltpu.SemaphoreType.DMA for _ in range(NB)],
        ],
        compiler_params=pltpu.CompilerParams(use_tc_tiling_on_sc=False),
    )
    def gather_kernel(w_hbm, table_hbm, out_hbm, idx_v, rows_v, gsem, ssem):
        wid = lax.axis_index("s") * NC + lax.axis_index("c")
        bbase = wid * BPW

        def fire_gathers(b, c):
            b0 = bbase + c * BBLK
            pltpu.sync_copy(w_hbm.at[pl.ds(b0, BBLK)], idx_v[b])
            for i in range(BBLK):
                pltpu.async_copy(
                    table_hbm.at[idx_v[b].at[i]], rows_v[b].at[i], gsem[b]
                )

        def wait_gathers(b):
            for i in range(BBLK):
                pltpu.make_async_copy(
                    table_hbm.at[idx_v[b].at[i]], rows_v[b].at[i], gsem[b]
                ).wait()

        def fire_store(b, c):
            pltpu.async_copy(
                rows_v[b], out_hbm.at[pl.ds(bbase + c * BBLK, BBLK)], ssem[b]
            )

        def wait_store(b, c):
            pltpu.make_async_copy(
                rows_v[b], out_hbm.at[pl.ds(bbase + c * BBLK, BBLK)], ssem[b]
            ).wait()

        for b in range(NB):
            fire_gathers(b, b)

        def body(g, carry):
            c0 = g * NB
            for b in range(NB):
                wait_gathers(b)
                fire_store(b, c0 + b)

            @pl.when(g < OUTER - 1)
            def _prefetch():
                for b in range(NB):
                    wait_store(b, c0 + b)
                    fire_gathers(b, c0 + NB + b)

            return carry

        lax.fori_loop(0, OUTER, body, 0)
        for b in range(NB):
            wait_store(b, (OUTER - 1) * NB + b)

    return gather_kernel


_gather = _make_gather()


@jax.jit
def kernel(w, table):
    return _gather(w.astype(jnp.int32), table)


# speedup vs baseline: 1.0191x; 1.0131x over previous
import functools

import jax
import jax.numpy as jnp
import numpy as np
from jax import lax
from jax.experimental import pallas as pl
from jax.experimental.pallas import tpu as pltpu
from jax.experimental.pallas import tpu_sc as plsc

BATCH = 16384
SEQ = 201
DIM = 64
BH = BATCH // 4
HB = BATCH // 2
HFLAT = HB * SEQ
NC, NS = 2, 16
NW = NC * NS
CHUNK = 256
IW = 128
IPC = CHUNK // IW
NCHUNKS = HFLAT // CHUNK
CHUNKS_PER_W = NCHUNKS // NW
CPS = HB // CHUNK
NB = 3
OUTER = CHUNKS_PER_W // NB

_j = np.arange(BATCH)
_PERM = jnp.asarray(
    (_j // 8192) * 8192 + ((_j % 8192) // 512) * 256
    + (_j % 512) // 2 + (_j % 2) * BH,
    dtype=jnp.int32,
)


def _make_gather():
    mesh = plsc.VectorSubcoreMesh(core_axis_name="c", subcore_axis_name="s")

    @functools.partial(
        pl.kernel,
        mesh=mesh,
        out_type=jax.ShapeDtypeStruct((HFLAT, DIM), jnp.float32),
        scratch_types=[
            [pltpu.VMEM((CHUNK,), jnp.int32) for _ in range(NB)],
            [pltpu.VMEM((CHUNK, DIM), jnp.float32) for _ in range(NB)],
            [pltpu.SemaphoreType.DMA for _ in range(NB)],
            [pltpu.SemaphoreType.DMA for _ in range(NB)],
        ],
        compiler_params=pltpu.CompilerParams(use_tc_tiling_on_sc=False),
    )
    def gather_kernel(wt_hbm, table_hbm, out_hbm, idx_v, rows_v, gsem, ssem):
        wid = lax.axis_index("s") * NC + lax.axis_index("c")
        base = wid * CHUNKS_PER_W

        def fire_gathers(b, c):
            cc = base + c
            s = cc // CPS
            off = (cc % CPS) * CHUNK
            pltpu.sync_copy(wt_hbm.at[s, pl.ds(off, CHUNK)], idx_v[b])
            for j in range(IPC):
                pltpu.async_copy(
                    table_hbm.at[idx_v[b].at[pl.ds(j * IW, IW)]],
                    rows_v[b].at[pl.ds(j * IW, IW)],
                    gsem[b],
                )

        def wait_gathers(b):
            for j in range(IPC):
                pltpu.make_async_copy(
                    table_hbm.at[idx_v[b].at[pl.ds(j * IW, IW)]],
                    rows_v[b].at[pl.ds(j * IW, IW)],
                    gsem[b],
                ).wait()

        def fire_store(b, c):
            pltpu.async_copy(
                rows_v[b], out_hbm.at[pl.ds((base + c) * CHUNK, CHUNK)], ssem[b]
            )

        def wait_store(b, c):
            pltpu.make_async_copy(
                rows_v[b], out_hbm.at[pl.ds((base + c) * CHUNK, CHUNK)], ssem[b]
            ).wait()

        for b in range(NB):
            fire_gathers(b, b)

        def body(g, carry):
            c0 = g * NB
            for b in range(NB):
                wait_gathers(b)
                fire_store(b, c0 + b)

            @pl.when(g < OUTER - 1)
            def _prefetch():
                for b in range(NB):
                    wait_store(b, c0 + b)
                    fire_gathers(b, c0 + NB + b)

            return carry

        lax.fori_loop(0, OUTER, body, 0)
        for b in range(NB):
            wait_store(b, (OUTER - 1) * NB + b)

    return gather_kernel


def _tr_kernel(x_ref, o_ref):
    x = x_ref[0]
    eye = jnp.eye(128, dtype=jnp.float32)
    yt = jax.lax.dot_general(
        eye, x, (((1,), (1,)), ((), ())),
        precision=jax.lax.Precision.HIGHEST,
        preferred_element_type=jnp.float32,
    )
    o_ref[0, :, :BH] = yt[:DIM]
    o_ref[0, :, BH:] = yt[DIM:]


_transpose = pl.pallas_call(
    _tr_kernel,
    grid=(SEQ,),
    in_specs=[pl.BlockSpec((1, BH, 128), lambda s: (s, 0, 0))],
    out_specs=pl.BlockSpec((1, DIM, 2 * BH), lambda s: (s, 0, 0)),
    out_shape=jax.ShapeDtypeStruct((SEQ, DIM, HB), jnp.float32),
)

_gather = _make_gather()


@jax.jit
def kernel(w, table):
    wp = w.astype(jnp.int32)[_PERM]
    wt0 = jnp.transpose(wp[:HB], (1, 0))
    wt1 = jnp.transpose(wp[HB:], (1, 0))
    oc0 = _gather(wt0, table)
    oc1 = _gather(wt1, table)
    ot0 = _transpose(oc0.reshape(SEQ, BH, 128))
    ot1 = _transpose(oc1.reshape(SEQ, BH, 128))
    ot = jnp.stack([ot0, ot1], axis=0)
    return jnp.transpose(ot, (0, 3, 1, 2)).reshape(BATCH, SEQ, DIM)


# speedup vs baseline: 1.6497x; 1.6189x over previous
import functools

import jax
import jax.numpy as jnp
import numpy as np
from jax import lax
from jax.experimental import pallas as pl
from jax.experimental.pallas import tpu as pltpu
from jax.experimental.pallas import tpu_sc as plsc

BATCH = 16384
SEQ = 201
DIM = 64
BH = BATCH // 4
BFLAT = BATCH * SEQ
NC, NS = 2, 16
NW = NC * NS
CHUNK = 512
IW = 128
IPC = CHUNK // IW
NCHUNKS = BFLAT // CHUNK
CHUNKS_PER_W = NCHUNKS // NW
CPS = BATCH // CHUNK
NB = 3
OUTER = CHUNKS_PER_W // NB

_j = np.arange(BATCH)
_PERM = jnp.asarray(
    (_j // 8192) * 8192 + ((_j % 8192) // 512) * 256
    + (_j % 512) // 2 + (_j % 2) * BH,
    dtype=jnp.int32,
)


def _make_gather():
    mesh = plsc.VectorSubcoreMesh(core_axis_name="c", subcore_axis_name="s")

    @functools.partial(
        pl.kernel,
        mesh=mesh,
        out_type=jax.ShapeDtypeStruct((BFLAT, DIM), jnp.float32),
        scratch_types=[
            [pltpu.VMEM((CHUNK,), jnp.int32) for _ in range(NB)],
            [pltpu.VMEM((CHUNK, DIM), jnp.float32) for _ in range(NB)],
            [pltpu.SemaphoreType.DMA for _ in range(NB)],
            [pltpu.SemaphoreType.DMA for _ in range(NB)],
        ],
        compiler_params=pltpu.CompilerParams(use_tc_tiling_on_sc=False),
    )
    def gather_kernel(wt_hbm, table_hbm, out_hbm, idx_v, rows_v, gsem, ssem):
        wid = lax.axis_index("s") * NC + lax.axis_index("c")
        base = wid * CHUNKS_PER_W

        def fire_gathers(b, c):
            cc = base + c
            s = cc // CPS
            off = (cc % CPS) * CHUNK
            pltpu.sync_copy(wt_hbm.at[s, pl.ds(off, CHUNK)], idx_v[b])
            for j in range(IPC):
                pltpu.async_copy(
                    table_hbm.at[idx_v[b].at[pl.ds(j * IW, IW)]],
                    rows_v[b].at[pl.ds(j * IW, IW)],
                    gsem[b],
                )

        def wait_gathers(b):
            for j in range(IPC):
                pltpu.make_async_copy(
                    table_hbm.at[idx_v[b].at[pl.ds(j * IW, IW)]],
                    rows_v[b].at[pl.ds(j * IW, IW)],
                    gsem[b],
                ).wait()

        def fire_store(b, c):
            pltpu.async_copy(
                rows_v[b], out_hbm.at[pl.ds((base + c) * CHUNK, CHUNK)], ssem[b]
            )

        def wait_store(b, c):
            pltpu.make_async_copy(
                rows_v[b], out_hbm.at[pl.ds((base + c) * CHUNK, CHUNK)], ssem[b]
            ).wait()

        for b in range(NB):
            fire_gathers(b, b)

        def body(g, carry):
            c0 = g * NB
            for b in range(NB):
                wait_gathers(b)
                fire_store(b, c0 + b)

            @pl.when(g < OUTER - 1)
            def _prefetch():
                for b in range(NB):
                    wait_store(b, c0 + b)
                    fire_gathers(b, c0 + NB + b)

            return carry

        lax.fori_loop(0, OUTER, body, 0)
        for b in range(NB):
            wait_store(b, (OUTER - 1) * NB + b)

    return gather_kernel


def _tr_kernel(x_ref, o_ref):
    x = x_ref[0]
    eye = jnp.eye(128, dtype=jnp.float32)
    yt = jax.lax.dot_general(
        eye, x, (((1,), (1,)), ((), ())),
        precision=jax.lax.Precision.HIGHEST,
        preferred_element_type=jnp.float32,
    )
    o_ref[0, :, :BH] = yt[:DIM]
    o_ref[0, :, BH:] = yt[DIM:]


_transpose = pl.pallas_call(
    _tr_kernel,
    grid=(SEQ, 2),
    in_specs=[pl.BlockSpec((1, BH, 128), lambda s, g: (2 * s + g, 0, 0))],
    out_specs=pl.BlockSpec((1, DIM, 2 * BH), lambda s, g: (s, 0, g)),
    out_shape=jax.ShapeDtypeStruct((SEQ, DIM, BATCH), jnp.float32),
)

_gather = _make_gather()


@jax.jit
def kernel(w, table):
    wp = w.astype(jnp.int32)[_PERM]
    wt = jnp.transpose(wp, (1, 0))
    oc = _gather(wt, table)
    ot = _transpose(oc.reshape(2 * SEQ, BH, 128))
    return jnp.transpose(ot, (2, 0, 1))
